# baseline (speedup 1.0000x reference)
import jax
import jax.numpy as jnp
from jax import lax
from jax.experimental import pallas as pl
from jax.experimental.pallas import tpu as pltpu
from jax.experimental.pallas import tpu_sc as plsc

NUM_CLS = 150
NBINS = NUM_CLS * NUM_CLS
HPAD = 22528
NC, NS, L = 2, 16, 16
NW = NC * NS
B, H, W = 16, 512, 512
ROWS_W = (B * H) // NW
CROWS = 32
CHUNK = CROWS * W
NCHUNK = ROWS_W // CROWS
NBUF = 2
UNROLL = 8


def _hist_body(p_hbm, t_hbm, out_hbm, p_buf0, p_buf1, t_buf0, t_buf1,
               hist0, hist1, sp0, sp1, st0, st1):
    wid = lax.axis_index("c") * NS + lax.axis_index("s")
    img = wid // 2
    row0 = (wid % 2) * (H // 2)

    zeros = jnp.zeros((L,), jnp.float32)

    @plsc.parallel_loop(0, HPAD, step=L, unroll=UNROLL)
    def _zero(o):
        hist0[o >> 7, pl.ds(o & 127, L)] = zeros
        hist1[o >> 7, pl.ds(o & 127, L)] = zeros

    ones = jnp.ones((L,), jnp.float32)
    pbufs = [p_buf0, p_buf1]
    tbufs = [t_buf0, t_buf1]
    sp = [sp0, sp1]
    st = [st0, st1]

    def start(i):
        r = row0 + i * CROWS
        s = i % NBUF
        dp = pltpu.async_copy(p_hbm.at[img, pl.ds(r, CROWS), :], pbufs[s], sp[s])
        dt = pltpu.async_copy(t_hbm.at[img, pl.ds(r, CROWS), :], tbufs[s], st[s])
        return dp, dt

    pend = [start(0)]
    for i in range(NCHUNK):
        if i + 1 < NCHUNK:
            pend.append(start(i + 1))
        dp, dt = pend[i]
        dp.wait()
        dt.wait()
        s = i % NBUF
        pb = pbufs[s]
        tb = tbufs[s]

        @plsc.parallel_loop(0, CHUNK, step=2 * L, unroll=UNROLL)
        def _inner(o, pb=pb, tb=tb):
            r0 = o >> 9
            c0 = o & (W - 1)
            p0 = pb[r0, pl.ds(c0, L)]
            t0 = tb[r0, pl.ds(c0, L)]
            idx0 = p0 * NUM_CLS + t0
            plsc.addupdate_scatter(hist0, [idx0 >> 7, idx0 & 127], ones)
            o1 = o + L
            r1 = o1 >> 9
            c1 = o1 & (W - 1)
            p1 = pb[r1, pl.ds(c1, L)]
            t1 = tb[r1, pl.ds(c1, L)]
            idx1 = p1 * NUM_CLS + t1
            plsc.addupdate_scatter(hist1, [idx1 >> 7, idx1 & 127], ones)

    pltpu.sync_copy(hist0, out_hbm.at[2 * wid])
    pltpu.sync_copy(hist1, out_hbm.at[2 * wid + 1])


@jax.jit
def _sc_hist(p, t):
    mesh = plsc.VectorSubcoreMesh(
        core_axis_name="c", subcore_axis_name="s",
        num_cores=NC, num_subcores=NS)
    f = pl.kernel(
        _hist_body,
        out_type=jax.ShapeDtypeStruct((2 * NW, HPAD // 128, 128), jnp.float32),
        mesh=mesh,
        compiler_params=pltpu.CompilerParams(
            needs_layout_passes=False, use_tc_tiling_on_sc=True),
        scratch_types=[
            pltpu.VMEM((CROWS, W), jnp.int32),
            pltpu.VMEM((CROWS, W), jnp.int32),
            pltpu.VMEM((CROWS, W), jnp.int32),
            pltpu.VMEM((CROWS, W), jnp.int32),
            pltpu.VMEM((HPAD // 128, 128), jnp.float32),
            pltpu.VMEM((HPAD // 128, 128), jnp.float32),
            pltpu.SemaphoreType.DMA,
            pltpu.SemaphoreType.DMA,
            pltpu.SemaphoreType.DMA,
            pltpu.SemaphoreType.DMA,
        ],
    )
    return f(p, t)


def kernel(preds, truths):
    parts = _sc_hist(preds, truths)
    acc = parts.sum(axis=0).reshape(HPAD)
    return acc[:NBINS].reshape(NUM_CLS, NUM_CLS)


# speedup vs baseline: 1.1030x; 1.1030x over previous
import jax
import jax.numpy as jnp
from jax import lax
from jax.experimental import pallas as pl
from jax.experimental.pallas import tpu as pltpu
from jax.experimental.pallas import tpu_sc as plsc

NUM_CLS = 150
NBINS = NUM_CLS * NUM_CLS
HPAD = 22528
NC, NS, L = 2, 16, 16
NW = NC * NS
B, H, W = 16, 512, 512
ROWS_W = (B * H) // NW
CROWS = 32
CHUNK = CROWS * W
NCHUNK = ROWS_W // CROWS
NBUF = 2
UNROLL = 8


def _hist_body(p_hbm, t_hbm, out_hbm, p_buf0, p_buf1, t_buf0, t_buf1, hist,
               sp0, sp1, st0, st1):
    wid = lax.axis_index("c") * NS + lax.axis_index("s")
    img = wid // 2
    row0 = (wid % 2) * (H // 2)

    pbufs = [p_buf0, p_buf1]
    tbufs = [t_buf0, t_buf1]
    sp = [sp0, sp1]
    st = [st0, st1]

    def start(i):
        r = row0 + i * CROWS
        s = i % NBUF
        dp = pltpu.async_copy(p_hbm.at[img, pl.ds(r, CROWS), :], pbufs[s], sp[s])
        dt = pltpu.async_copy(t_hbm.at[img, pl.ds(r, CROWS), :], tbufs[s], st[s])
        return dp, dt

    pend = [start(0)]

    zeros = jnp.zeros((L,), jnp.float32)

    @plsc.parallel_loop(0, HPAD, step=L, unroll=UNROLL)
    def _zero(o):
        hist[o >> 7, pl.ds(o & 127, L)] = zeros

    ones = jnp.ones((L,), jnp.float32)
    for i in range(NCHUNK):
        if i + 1 < NCHUNK:
            pend.append(start(i + 1))
        dp, dt = pend[i]
        dp.wait()
        dt.wait()
        s = i % NBUF
        pb = pbufs[s]
        tb = tbufs[s]

        @plsc.parallel_loop(0, CHUNK, step=L, unroll=UNROLL)
        def _inner(o, pb=pb, tb=tb):
            r = o >> 9
            c = o & (W - 1)
            p = pb[r, pl.ds(c, L)]
            t = tb[r, pl.ds(c, L)]
            idx = p * NUM_CLS + t
            plsc.addupdate_scatter(hist, [idx >> 7, idx & 127], ones)

    pltpu.sync_copy(hist, out_hbm.at[wid])


@jax.jit
def _sc_hist(p, t):
    mesh = plsc.VectorSubcoreMesh(
        core_axis_name="c", subcore_axis_name="s",
        num_cores=NC, num_subcores=NS)
    f = pl.kernel(
        _hist_body,
        out_type=jax.ShapeDtypeStruct((NW, HPAD // 128, 128), jnp.float32),
        mesh=mesh,
        compiler_params=pltpu.CompilerParams(
            needs_layout_passes=False, use_tc_tiling_on_sc=True),
        scratch_types=[
            pltpu.VMEM((CROWS, W), jnp.int32),
            pltpu.VMEM((CROWS, W), jnp.int32),
            pltpu.VMEM((CROWS, W), jnp.int32),
            pltpu.VMEM((CROWS, W), jnp.int32),
            pltpu.VMEM((HPAD // 128, 128), jnp.float32),
            pltpu.SemaphoreType.DMA,
            pltpu.SemaphoreType.DMA,
            pltpu.SemaphoreType.DMA,
            pltpu.SemaphoreType.DMA,
        ],
    )
    return f(p, t)


def kernel(preds, truths):
    parts = _sc_hist(preds, truths)
    acc = parts.sum(axis=0).reshape(HPAD)
    return acc[:NBINS].reshape(NUM_CLS, NUM_CLS)
